# Initial kernel scaffold; baseline (speedup 1.0000x reference)
import jax
import jax.numpy as jnp
from jax.experimental import pallas as pl


def kernel(context, embedding, W, b):
    raise NotImplementedError("write your pallas kernel here")


# speedup vs baseline: 8.6004x; 8.6004x over previous
import functools

import jax
import jax.numpy as jnp
from jax import lax
from jax.experimental import pallas as pl
from jax.experimental.pallas import tpu as pltpu
from jax.experimental.pallas import tpu_sc as plsc

VOCAB = 65
EMBED = 128
BATCH = 16384
CTX = 10
PADW = 80

NC = 2
NS = 16
NW = NC * NS
ROWS_PER = BATCH // NW


def _fuse_table_body(emb_ref, w_ref, b_ref, out_ref):
    m = lax.dot_general(
        emb_ref[...], w_ref[...],
        (((1,), (1,)), ((), ())),
        preferred_element_type=jnp.float32,
    )
    out_ref[...] = (m + b_ref[...]) * (1.0 / CTX)


def _build_table(embedding, W, b):
    w_pad = jnp.zeros((PADW, EMBED), jnp.float32).at[:VOCAB].set(W)
    b_pad = jnp.zeros((1, PADW), jnp.float32).at[0, :VOCAB].set(b)
    return pl.pallas_call(
        _fuse_table_body,
        out_shape=jax.ShapeDtypeStruct((VOCAB, PADW), jnp.float32),
    )(embedding, w_pad, b_pad)


_sc_mesh = plsc.VectorSubcoreMesh(core_axis_name="c", subcore_axis_name="s")


@functools.partial(
    pl.kernel,
    mesh=_sc_mesh,
    out_type=jax.ShapeDtypeStruct((BATCH * VOCAB,), jnp.float32),
    scratch_types=[
        pltpu.VMEM((ROWS_PER * CTX + 16,), jnp.int32),
        pltpu.VMEM((VOCAB, PADW), jnp.float32),
        pltpu.VMEM((ROWS_PER * VOCAB + 16,), jnp.float32),
    ],
)
def _sc_pool(ctx_hbm, tab_hbm, out_hbm, ctx_v, tab_v, out_v):
    wid = lax.axis_index("s") * NC + lax.axis_index("c")
    base = wid * ROWS_PER
    pltpu.sync_copy(
        ctx_hbm.at[pl.ds(base * CTX, ROWS_PER * CTX)],
        ctx_v.at[pl.ds(0, ROWS_PER * CTX)],
    )
    pltpu.sync_copy(tab_hbm, tab_v)

    def body(i, _):
        idxs = ctx_v[pl.ds(i * CTX, 16)]
        idx0 = idxs[0]
        accs = [tab_v[idx0, pl.ds(16 * k, 16)] for k in range(5)]
        for c in range(1, CTX):
            idx = idxs[c]
            for k in range(5):
                accs[k] = accs[k] + tab_v[idx, pl.ds(16 * k, 16)]
        rb = i * VOCAB
        for k in range(5):
            out_v[pl.ds(rb + 16 * k, 16)] = accs[k]
        return _

    lax.fori_loop(0, ROWS_PER, body, None)
    pltpu.sync_copy(
        out_v.at[pl.ds(0, ROWS_PER * VOCAB)],
        out_hbm.at[pl.ds(base * VOCAB, ROWS_PER * VOCAB)],
    )


def kernel(context, embedding, W, b):
    table = _build_table(embedding, W, b)
    out_flat = _sc_pool(context.reshape(BATCH * CTX), table)
    return out_flat.reshape(BATCH, VOCAB)


# speedup vs baseline: 8.6717x; 1.0083x over previous
import functools

import jax
import jax.numpy as jnp
from jax import lax
from jax.experimental import pallas as pl
from jax.experimental.pallas import tpu as pltpu
from jax.experimental.pallas import tpu_sc as plsc

VOCAB = 65
EMBED = 128
BATCH = 16384
CTX = 10
PADW = 80

NC = 2
NS = 16
NW = NC * NS
ROWS_PER = BATCH // NW


def _fuse_table_body(emb_ref, w_ref, b_ref, out_ref):
    m = lax.dot_general(
        emb_ref[...], w_ref[...],
        (((1,), (1,)), ((), ())),
        preferred_element_type=jnp.float32,
    )
    out_ref[...] = (m + b_ref[...]) * (1.0 / CTX)


def _build_table(embedding, W, b):
    w_pad = jnp.zeros((PADW, EMBED), jnp.float32).at[:VOCAB].set(W)
    b_pad = jnp.zeros((1, PADW), jnp.float32).at[0, :VOCAB].set(b)
    return pl.pallas_call(
        _fuse_table_body,
        out_shape=jax.ShapeDtypeStruct((VOCAB, PADW), jnp.float32),
    )(embedding, w_pad, b_pad)


_sc_mesh = plsc.VectorSubcoreMesh(core_axis_name="c", subcore_axis_name="s")


@functools.partial(
    pl.kernel,
    mesh=_sc_mesh,
    out_type=jax.ShapeDtypeStruct((BATCH * VOCAB,), jnp.float32),
    scratch_types=[
        pltpu.VMEM((ROWS_PER * CTX + 16,), jnp.int32),
        pltpu.VMEM((VOCAB, PADW), jnp.float32),
        pltpu.VMEM((ROWS_PER * VOCAB + 16,), jnp.float32),
    ],
)
def _sc_pool(ctx_hbm, tab_hbm, out_hbm, ctx_v, tab_v, out_v):
    wid = lax.axis_index("s") * NC + lax.axis_index("c")
    base = wid * ROWS_PER
    pltpu.sync_copy(
        ctx_hbm.at[pl.ds(base * CTX, ROWS_PER * CTX)],
        ctx_v.at[pl.ds(0, ROWS_PER * CTX)],
    )
    pltpu.sync_copy(tab_hbm, tab_v)

    def body(i, _):
        idxs = ctx_v[pl.ds(i * CTX, 16)]
        idx0 = idxs[0]
        accs = [tab_v[idx0, pl.ds(16 * k, 16)] for k in range(5)]
        for c in range(1, CTX):
            idx = idxs[c]
            for k in range(5):
                accs[k] = accs[k] + tab_v[idx, pl.ds(16 * k, 16)]
        rb = i * VOCAB
        for k in range(5):
            out_v[pl.ds(rb + 16 * k, 16)] = accs[k]
        return _

    lax.fori_loop(0, ROWS_PER, body, None, unroll=4)
    pltpu.sync_copy(
        out_v.at[pl.ds(0, ROWS_PER * VOCAB)],
        out_hbm.at[pl.ds(base * VOCAB, ROWS_PER * VOCAB)],
    )


def kernel(context, embedding, W, b):
    table = _build_table(embedding, W, b)
    out_flat = _sc_pool(context.reshape(BATCH * CTX), table)
    return out_flat.reshape(BATCH, VOCAB)
